# baseline (speedup 1.0000x reference)
import functools

import jax
import jax.numpy as jnp
from jax import lax
from jax.experimental import pallas as pl
from jax.experimental.pallas import tpu as pltpu
from jax.experimental.pallas import tpu_sc as plsc

_NBUF = 8
_TCH = 800

_PARAMS = pltpu.CompilerParams(use_tc_tiling_on_sc=False,
                               needs_layout_passes=False)


@functools.cache
def _sc_info():
    info = plsc.get_sparse_core_info()
    return info.num_cores, info.num_subcores, info.num_lanes


@functools.cache
def _build_transpose(V, D):
    NC, NS, NL = _sc_info()
    NW = NC * NS
    VTILE = (V // 128) * 128
    TAIL = V - VTILE
    CPC = 4
    CH = CPC * 128
    NCHUNK = (VTILE // 128) // CPC
    TMAX = -(-NCHUNK // NW)
    TMAX += (-TMAX) % 4
    NTR = D // 8
    assert D == 2 * NL

    mesh = plsc.VectorSubcoreMesh(core_axis_name="c", subcore_axis_name="s")

    @functools.partial(
        pl.kernel,
        mesh=mesh,
        out_type=jax.ShapeDtypeStruct((V * D,), jnp.float32),
        scratch_types=[
            pltpu.VMEM((2, NTR, 8, CH), jnp.float32),
            pltpu.VMEM((CH * D,), jnp.float32),
            pltpu.VMEM((CH * D,), jnp.float32),
            pltpu.VMEM((CH * D,), jnp.float32),
            pltpu.VMEM((CH * D,), jnp.float32),
            pltpu.VMEM((max(TAIL, 1) * D,), jnp.float32),
            [pltpu.SemaphoreType.DMA] * 2,
            [pltpu.SemaphoreType.DMA] * 4,
        ],
        compiler_params=pltpu.CompilerParams(use_tc_tiling_on_sc=True,
                                             needs_layout_passes=False),
    )
    def body(wt_hbm, tail_hbm, out_hbm, stg, ob0, ob1, ob2, ob3, tbuf,
             sin, sout):
        obs = [ob0, ob1, ob2, ob3]
        wid = lax.axis_index("s") * NC + lax.axis_index("c")
        iota = lax.iota(jnp.int32, NL)
        ci_lo = iota // 8
        ci_hi = ci_lo + 2
        cd = iota % 8
        rot = [(iota + k) % NL for k in range(NL)]
        sct = [r * D + iota for r in rot]

        @pl.when(wid == 0)
        def _():
            if TAIL:
                pltpu.sync_copy(tail_hbm, tbuf)
                pltpu.sync_copy(tbuf, out_hbm.at[pl.ds(VTILE * D, TAIL * D)])

        def in_copies(c, s):
            return [pltpu.make_async_copy(
                wt_hbm.at[pl.ds(8 * i, 8), pl.ds(c * CH, CH)],
                stg.at[s, i], sin[s]) for i in range(NTR)]

        def out_copy(c, b):
            return pltpu.make_async_copy(
                obs[b], out_hbm.at[pl.ds(c * CH * D, CH * D)], sout[b])

        @pl.when(wid < NCHUNK)
        def _():
            for cp in in_copies(wid, 0):
                cp.start()

        def t_body(tt, carry):
            for u in range(4):
                t = tt * 4 + u
                c = t * NW + wid
                s = u % 2
                ob = obs[u]

                @pl.when(c < NCHUNK)
                def _():
                    for cp in in_copies(c, s):
                        cp.wait()

                    @pl.when(c + NW < NCHUNK)
                    def _():
                        for cp in in_copies(c + NW, 1 - s):
                            cp.start()

                    @pl.when(c >= 4 * NW)
                    def _():
                        out_copy(c - 4 * NW, u).wait()

                    def blk_body(b0, carry2):
                        for half in range(2):
                            v0 = (b0 * 2 + half) * NL
                            base = v0 * D
                            for k in range(NL):
                                col = rot[k] + v0
                                g1 = plsc.load_gather(stg.at[s],
                                                      [ci_lo, cd, col])
                                g2 = plsc.load_gather(stg.at[s],
                                                      [ci_hi, cd, col])
                                plsc.store_scatter(ob, [sct[k] + base], g1)
                                plsc.store_scatter(
                                    ob, [sct[k] + (base + NL)], g2)
                        return carry2

                    lax.fori_loop(0, CH // NL // 2, blk_body, 0)

                    @pl.when(c >= NW)
                    def _():
                        out_copy(c - NW, (u + 3) % 4).start()

                    @pl.when(c + NW >= NCHUNK)
                    def _():
                        cp = out_copy(c, u)
                        cp.start()
                        cp.wait()
            return carry

        lax.fori_loop(0, TMAX // 4, t_body, 0)

        nv = (NCHUNK - wid + NW - 1) // NW
        for b in range(4):
            qb = (nv - 2) - ((nv - 2 - b) % 4)

            @pl.when((qb >= nv - 4) & (qb >= 0))
            def _():
                out_copy(qb * NW + wid, b).wait()

    return body


@functools.cache
def _build_gather(B, L, V, D):
    NC, NS, NL = _sc_info()
    NW = NC * NS
    B_PER = B // NW
    GIDX = 2 * L
    NGRP = B_PER // 2
    LC = -(-L // NL)
    assert GIDX <= 128 and D % NL == 0 and NGRP % _NBUF == 0
    n_vec = D // NL

    mesh = plsc.VectorSubcoreMesh(core_axis_name="c", subcore_axis_name="s")

    @functools.partial(
        pl.kernel,
        mesh=mesh,
        out_type=jax.ShapeDtypeStruct((D, B), jnp.float32),
        scratch_types=[
            pltpu.VMEM((LC * NL, B_PER + 1), jnp.int32),
            pltpu.VMEM((NGRP, GIDX), jnp.int32),
            pltpu.VMEM((_NBUF, GIDX, D), jnp.float32),
            pltpu.VMEM((D, B_PER + 1), jnp.float32),
            [pltpu.SemaphoreType.DMA] * _NBUF,
        ],
        compiler_params=_PARAMS,
    )
    def body(xt_hbm, table_hbm, out_hbm, stage, idx_v, bufs, out_t, sems):
        wid = lax.axis_index("s") * NC + lax.axis_index("c")
        col0 = wid * B_PER
        pltpu.sync_copy(xt_hbm.at[:, pl.ds(col0, B_PER)],
                        stage.at[pl.ds(0, L), pl.ds(0, B_PER)])

        iota = lax.iota(jnp.int32, NL)

        def transpose_body(b, carry):
            row = jnp.full((NL,), b // 2, jnp.int32)
            cbase = (b % 2) * L
            for jc in range(LC):
                j = jc * NL + iota
                vals = plsc.load_gather(
                    stage, [j, jnp.full((NL,), b, jnp.int32)])
                if (jc + 1) * NL <= L:
                    plsc.store_scatter(idx_v, [row, cbase + j], vals)
                else:
                    plsc.store_scatter(idx_v, [row, cbase + j], vals,
                                       mask=j < L)
            return carry

        lax.fori_loop(0, B_PER, transpose_body, 0)

        def start(c, b):
            pltpu.async_copy(table_hbm.at[idx_v.at[c]], bufs.at[b], sems[b])

        def drain(c, b):
            pltpu.make_async_copy(
                table_hbm.at[idx_v.at[c]], bufs.at[b], sems[b]
            ).wait()

        def reduce_group(c, b):
            for k in range(2):
                base = k * L
                accs = [bufs[b, base, pl.ds(v * NL, NL)]
                        for v in range(n_vec)]
                for j in range(1, L):
                    for v in range(n_vec):
                        accs[v] += bufs[b, base + j, pl.ds(v * NL, NL)]
                r = jnp.full((NL,), c * 2 + k, jnp.int32)
                for v in range(n_vec):
                    plsc.store_scatter(out_t, [v * NL + iota, r],
                                       accs[v] * (1.0 / L))

        for b in range(_NBUF):
            start(b, b)

        def loop_body(g, carry):
            for b in range(_NBUF):
                c = g * _NBUF + b
                drain(c, b)
                reduce_group(c, b)

                @pl.when(g < NGRP // _NBUF - 1)
                def _():
                    start(c + _NBUF, b)
            return carry

        lax.fori_loop(0, NGRP // _NBUF, loop_body, 0)
        pltpu.sync_copy(out_t.at[:, pl.ds(0, B_PER)],
                        out_hbm.at[:, pl.ds(col0, B_PER)])

    return body


def kernel(x, W):
    B, L = x.shape
    V, D = W.shape
    VTILE = (V // 128) * 128
    tail = W[VTILE:].reshape(-1)
    table = _build_transpose(V, D)(W.T, tail).reshape(V, D)
    out_t = _build_gather(B, L, V, D)(x.T.astype(jnp.int32), table)
    return out_t.T


# speedup vs baseline: 1.1688x; 1.1688x over previous
import functools

import jax
import jax.numpy as jnp
from jax import lax
from jax.experimental import pallas as pl
from jax.experimental.pallas import tpu as pltpu
from jax.experimental.pallas import tpu_sc as plsc

_NBUF = 4
_TCH = 800

_PARAMS = pltpu.CompilerParams(use_tc_tiling_on_sc=False,
                               needs_layout_passes=False)


@functools.cache
def _sc_info():
    info = plsc.get_sparse_core_info()
    return info.num_cores, info.num_subcores, info.num_lanes


@functools.cache
def _build_transpose(V, D):
    NC, NS, NL = _sc_info()
    NW = NC * NS
    VTILE = (V // 128) * 128
    TAIL = V - VTILE
    CPC = 4
    CH = CPC * 128
    NCHUNK = (VTILE // 128) // CPC
    TMAX = -(-NCHUNK // NW)
    TMAX += (-TMAX) % 4
    NTR = D // 8
    assert D == 2 * NL

    mesh = plsc.VectorSubcoreMesh(core_axis_name="c", subcore_axis_name="s")

    @functools.partial(
        pl.kernel,
        mesh=mesh,
        out_type=jax.ShapeDtypeStruct((V * D,), jnp.float32),
        scratch_types=[
            pltpu.VMEM((2, NTR, 8, CH), jnp.float32),
            pltpu.VMEM((CH * D,), jnp.float32),
            pltpu.VMEM((CH * D,), jnp.float32),
            pltpu.VMEM((CH * D,), jnp.float32),
            pltpu.VMEM((CH * D,), jnp.float32),
            pltpu.VMEM((max(TAIL, 1) * D,), jnp.float32),
            [pltpu.SemaphoreType.DMA] * 2,
            [pltpu.SemaphoreType.DMA] * 4,
        ],
        compiler_params=pltpu.CompilerParams(use_tc_tiling_on_sc=True,
                                             needs_layout_passes=False),
    )
    def body(wt_hbm, tail_hbm, out_hbm, stg, ob0, ob1, ob2, ob3, tbuf,
             sin, sout):
        obs = [ob0, ob1, ob2, ob3]
        wid = lax.axis_index("s") * NC + lax.axis_index("c")
        iota = lax.iota(jnp.int32, NL)
        ci_lo = iota // 8
        ci_hi = ci_lo + 2
        cd = iota % 8
        rot = [(iota + k) % NL for k in range(NL)]
        sct = [r * D + iota for r in rot]

        @pl.when(wid == 0)
        def _():
            if TAIL:
                pltpu.sync_copy(tail_hbm, tbuf)
                pltpu.sync_copy(tbuf, out_hbm.at[pl.ds(VTILE * D, TAIL * D)])

        def in_copies(c, s):
            return [pltpu.make_async_copy(
                wt_hbm.at[pl.ds(8 * i, 8), pl.ds(c * CH, CH)],
                stg.at[s, i], sin[s]) for i in range(NTR)]

        def out_copy(c, b):
            return pltpu.make_async_copy(
                obs[b], out_hbm.at[pl.ds(c * CH * D, CH * D)], sout[b])

        @pl.when(wid < NCHUNK)
        def _():
            for cp in in_copies(wid, 0):
                cp.start()

        def t_body(tt, carry):
            for u in range(4):
                t = tt * 4 + u
                c = t * NW + wid
                s = u % 2
                ob = obs[u]

                @pl.when(c < NCHUNK)
                def _():
                    for cp in in_copies(c, s):
                        cp.wait()

                    @pl.when(c + NW < NCHUNK)
                    def _():
                        for cp in in_copies(c + NW, 1 - s):
                            cp.start()

                    @pl.when(c >= 4 * NW)
                    def _():
                        out_copy(c - 4 * NW, u).wait()

                    def blk_body(b0, carry2):
                        for half in range(4):
                            v0 = (b0 * 4 + half) * NL
                            base = v0 * D
                            for k in range(NL):
                                col = rot[k] + v0
                                g1 = plsc.load_gather(stg.at[s],
                                                      [ci_lo, cd, col])
                                g2 = plsc.load_gather(stg.at[s],
                                                      [ci_hi, cd, col])
                                plsc.store_scatter(ob, [sct[k] + base], g1)
                                plsc.store_scatter(
                                    ob, [sct[k] + (base + NL)], g2)
                        return carry2

                    lax.fori_loop(0, CH // NL // 4, blk_body, 0)

                    @pl.when(c >= NW)
                    def _():
                        out_copy(c - NW, (u + 3) % 4).start()

                    @pl.when(c + NW >= NCHUNK)
                    def _():
                        cp = out_copy(c, u)
                        cp.start()
                        cp.wait()
            return carry

        lax.fori_loop(0, TMAX // 4, t_body, 0)

        nv = (NCHUNK - wid + NW - 1) // NW
        for b in range(4):
            qb = (nv - 2) - ((nv - 2 - b) % 4)

            @pl.when((qb >= nv - 4) & (qb >= 0))
            def _():
                out_copy(qb * NW + wid, b).wait()

    return body


@functools.cache
def _build_gather(B, L, V, D):
    NC, NS, NL = _sc_info()
    NW = NC * NS
    B_PER = B // NW
    GIDX = 2 * L
    NGRP = B_PER // 2
    LC = -(-L // NL)
    assert GIDX <= 128 and D % NL == 0 and NGRP % _NBUF == 0
    n_vec = D // NL

    mesh = plsc.VectorSubcoreMesh(core_axis_name="c", subcore_axis_name="s")

    @functools.partial(
        pl.kernel,
        mesh=mesh,
        out_type=jax.ShapeDtypeStruct((D, B), jnp.float32),
        scratch_types=[
            pltpu.VMEM((LC * NL, B_PER + 1), jnp.int32),
            pltpu.VMEM((NGRP, GIDX), jnp.int32),
            pltpu.VMEM((_NBUF, GIDX, D), jnp.float32),
            pltpu.VMEM((D, B_PER + 1), jnp.float32),
            [pltpu.SemaphoreType.DMA] * _NBUF,
        ],
        compiler_params=_PARAMS,
    )
    def body(xt_hbm, table_hbm, out_hbm, stage, idx_v, bufs, out_t, sems):
        wid = lax.axis_index("s") * NC + lax.axis_index("c")
        col0 = wid * B_PER
        pltpu.sync_copy(xt_hbm.at[:, pl.ds(col0, B_PER)],
                        stage.at[pl.ds(0, L), pl.ds(0, B_PER)])

        iota = lax.iota(jnp.int32, NL)

        def transpose_body(b, carry):
            row = jnp.full((NL,), b // 2, jnp.int32)
            cbase = (b % 2) * L
            for jc in range(LC):
                j = jc * NL + iota
                vals = plsc.load_gather(
                    stage, [j, jnp.full((NL,), b, jnp.int32)])
                if (jc + 1) * NL <= L:
                    plsc.store_scatter(idx_v, [row, cbase + j], vals)
                else:
                    plsc.store_scatter(idx_v, [row, cbase + j], vals,
                                       mask=j < L)
            return carry

        lax.fori_loop(0, B_PER, transpose_body, 0)

        def start(c, b):
            pltpu.async_copy(table_hbm.at[idx_v.at[c]], bufs.at[b], sems[b])

        def drain(c, b):
            pltpu.make_async_copy(
                table_hbm.at[idx_v.at[c]], bufs.at[b], sems[b]
            ).wait()

        def reduce_group(c, b):
            for k in range(2):
                base = k * L
                accs = [bufs[b, base, pl.ds(v * NL, NL)]
                        for v in range(n_vec)]
                for j in range(1, L):
                    for v in range(n_vec):
                        accs[v] += bufs[b, base + j, pl.ds(v * NL, NL)]
                r = jnp.full((NL,), c * 2 + k, jnp.int32)
                for v in range(n_vec):
                    plsc.store_scatter(out_t, [v * NL + iota, r],
                                       accs[v] * (1.0 / L))

        for b in range(_NBUF):
            start(b, b)

        def loop_body(g, carry):
            for b in range(_NBUF):
                c = g * _NBUF + b
                drain(c, b)
                reduce_group(c, b)

                @pl.when(g < NGRP // _NBUF - 1)
                def _():
                    start(c + _NBUF, b)
            return carry

        lax.fori_loop(0, NGRP // _NBUF, loop_body, 0)
        pltpu.sync_copy(out_t.at[:, pl.ds(0, B_PER)],
                        out_hbm.at[:, pl.ds(col0, B_PER)])

    return body


def kernel(x, W):
    B, L = x.shape
    V, D = W.shape
    VTILE = (V // 128) * 128
    tail = W[VTILE:].reshape(-1)
    table = _build_transpose(V, D)(W.T, tail).reshape(V, D)
    out_t = _build_gather(B, L, V, D)(x.T.astype(jnp.int32), table)
    return out_t.T
